# baseline (speedup 1.0000x reference)
import functools

import jax
import jax.numpy as jnp
from jax import lax
from jax.experimental import pallas as pl
from jax.experimental.pallas import tpu as pltpu
from jax.experimental.pallas import tpu_sc as plsc

RES = 512
NCH = 32
LANES = 16
CHUNK = 128
GROUPS = CHUNK // LANES
NCORES = 2
NSUB = 16
NW = NCORES * NSUB

PLANE_COLS = ((0, 1), (0, 2), (1, 2))


@functools.lru_cache(maxsize=None)
def _build(npts):
    per_w = npts // NW
    nchunks = per_w // CHUNK
    assert nchunks % 2 == 0
    mesh = plsc.VectorSubcoreMesh(core_axis_name="c", subcore_axis_name="s")

    @functools.partial(
        pl.kernel,
        mesh=mesh,
        out_type=jax.ShapeDtypeStruct((npts, 3 * NCH), jnp.float32),
        compiler_params=pltpu.CompilerParams(
            needs_layout_passes=False, use_tc_tiling_on_sc=False),
        scratch_types=(
            [pltpu.VMEM((3 * RES * NCH,), jnp.float32)]
            + [pltpu.VMEM((CHUNK, 3), jnp.float32)
               for _ in range(2)]
            + [pltpu.VMEM((CHUNK,), jnp.int32)
               for _ in range(24)]
            + [pltpu.VMEM((CHUNK,), jnp.float32)
               for _ in range(36)]
            + [pltpu.VMEM((CHUNK,), jnp.int32)
               for _ in range(6)]
            + [pltpu.VMEM((CHUNK, NCH), jnp.float32)
               for _ in range(8)]
            + [pltpu.VMEM((CHUNK, 3 * NCH), jnp.float32)
               for _ in range(2)]
            + [pltpu.SemaphoreType.DMA] * 6
        ),
    )
    def enc(x_hbm, planes_hbm, lines_hbm, out_hbm, *scratch):
        line_v = scratch[0]
        xbuf = [scratch[1 + j] for j in range(2)]
        idx = [scratch[3 + j] for j in range(24)]
        wgt = [scratch[27 + j] for j in range(36)]
        lbv = [scratch[63 + j] for j in range(6)]
        rbuf = [scratch[69 + j] for j in range(8)]
        outb = [scratch[77 + j] for j in range(2)]
        semA, semB, semX0, semX1, semO0, semO1 = scratch[79:85]
        semX = [semX0, semX1]
        semO = [semO0, semO1]

        wid = lax.axis_index("s") * NCORES + lax.axis_index("c")
        wbase = wid * per_w
        pltpu.sync_copy(lines_hbm, line_v)
        iota = lax.iota(jnp.int32, LANES)

        def fire_x(ci, par):
            return pltpu.async_copy(
                x_hbm.at[pl.ds(wbase + ci * CHUNK, CHUNK)], xbuf[par],
                semX[par])

        def fire_planes(k, s, ab, sem):
            bufs = rbuf[ab * 4:ab * 4 + 4]
            return [pltpu.async_copy(
                planes_hbm.at[k].at[idx[s * 12 + k * 4 + j]],
                bufs[j], sem) for j in range(4)]

        def wait_planes(k, s, ab, sem):
            bufs = rbuf[ab * 4:ab * 4 + 4]
            for j in range(4):
                pltpu.make_async_copy(
                    planes_hbm.at[k].at[idx[s * 12 + k * 4 + j]], bufs[j],
                    sem).wait()

        def grp_idx(s, xv):
            def body(g, _):
                rows = g * LANES + iota
                sl = pl.ds(g * LANES, LANES)
                i0 = []
                fr = []
                for col in range(3):
                    xc = plsc.load_gather(
                        xv, [rows, jnp.full((LANES,), col, jnp.int32)])
                    ic = xc * float(RES - 1)
                    ic0 = jnp.minimum(ic.astype(jnp.int32), RES - 2)
                    i0.append(ic0)
                    fr.append(ic - ic0.astype(jnp.float32))
                for k in range(3):
                    axc, ayc = PLANE_COLS[k]
                    fx, fy, fl = fr[axc], fr[ayc], fr[2 - k]
                    gx = 1.0 - fx
                    gy = 1.0 - fy
                    wgt[s * 18 + k * 6 + 0][sl] = gx * gy
                    wgt[s * 18 + k * 6 + 1][sl] = fx * gy
                    wgt[s * 18 + k * 6 + 2][sl] = gx * fy
                    wgt[s * 18 + k * 6 + 3][sl] = fx * fy
                    wgt[s * 18 + k * 6 + 4][sl] = 1.0 - fl
                    wgt[s * 18 + k * 6 + 5][sl] = fl
                    row00 = i0[ayc] * RES + i0[axc]
                    idx[s * 12 + k * 4 + 0][sl] = row00
                    idx[s * 12 + k * 4 + 1][sl] = row00 + 1
                    idx[s * 12 + k * 4 + 2][sl] = row00 + RES
                    idx[s * 12 + k * 4 + 3][sl] = row00 + RES + 1
                    lbv[s * 3 + k][sl] = (k * RES + i0[2 - k]) * NCH
                return 0

            lax.fori_loop(0, GROUPS, body, 0)

        def compute(k, s, ab, out_v):
            r00, r01, r10, r11 = rbuf[ab * 4:ab * 4 + 4]
            w = wgt[s * 18 + k * 6:s * 18 + k * 6 + 6]
            lb_v = lbv[s * 3 + k]

            @plsc.parallel_loop(0, GROUPS, unroll=4)
            def grp(g):
                gbase = g * LANES
                gsl = pl.ds(gbase, LANES)
                wg = [w[i][gsl] for i in range(6)]
                lbg = lb_v[gsl]
                for j in range(LANES):
                    p = gbase + j
                    w00 = jnp.full((LANES,), wg[0][j], jnp.float32)
                    w01 = jnp.full((LANES,), wg[1][j], jnp.float32)
                    w10 = jnp.full((LANES,), wg[2][j], jnp.float32)
                    w11 = jnp.full((LANES,), wg[3][j], jnp.float32)
                    wl0 = jnp.full((LANES,), wg[4][j], jnp.float32)
                    wl1 = jnp.full((LANES,), wg[5][j], jnp.float32)
                    lb = lbg[j]
                    for h in range(2):
                        off = h * LANES
                        sl = pl.ds(off, LANES)
                        v00 = r00[p, sl]
                        v01 = r01[p, sl]
                        v10 = r10[p, sl]
                        v11 = r11[p, sl]
                        l0 = line_v[pl.ds(lb + off, LANES)]
                        l1 = line_v[pl.ds(lb + NCH + off, LANES)]
                        pf = w00 * v00 + w01 * v01 + w10 * v10 + w11 * v11
                        lf = wl0 * l0 + wl1 * l1
                        out_v[p, pl.ds(k * NCH + off, LANES)] = pf * lf

        fire_x(0, 0).wait()
        grp_idx(0, xbuf[0])
        fire_planes(0, 0, 0, semA)
        fire_planes(1, 0, 1, semB)
        fire_x(1, 1)

        def half(ci, par):
            s = par
            nxt = jnp.minimum(ci + 1, nchunks - 1)
            base = wbase + ci * CHUNK
            out_v = outb[par]

            @pl.when(ci >= 2)
            def _():
                pltpu.make_async_copy(
                    out_v, out_hbm.at[pl.ds(base, CHUNK)], semO[par]).wait()

            wait_planes(0, s, 0, semA)
            compute(0, s, 0, out_v)
            fire_planes(2, s, 0, semA)
            wait_planes(1, s, 1, semB)
            compute(1, s, 1, out_v)
            wait_planes(2, s, 0, semA)
            compute(2, s, 0, out_v)

            pltpu.make_async_copy(
                x_hbm.at[pl.ds(wbase + nxt * CHUNK, CHUNK)], xbuf[1 - par],
                semX[1 - par]).wait()
            grp_idx(1 - s, xbuf[1 - par])
            fire_planes(0, 1 - s, 0, semA)
            fire_planes(1, 1 - s, 1, semB)
            nxt2 = jnp.minimum(ci + 2, nchunks - 1)
            fire_x(nxt2, par)
            pltpu.async_copy(out_v, out_hbm.at[pl.ds(base, CHUNK)], semO[par])

        def pair(pi, _):
            half(2 * pi, 0)
            half(2 * pi + 1, 1)
            return 0

        lax.fori_loop(0, nchunks // 2, pair, 0)

        lastb = wbase + (nchunks - 1) * CHUNK
        wait_planes(0, 0, 0, semA)
        wait_planes(1, 0, 1, semB)
        pltpu.make_async_copy(
            x_hbm.at[pl.ds(lastb, CHUNK)], xbuf[1], semX[1]).wait()
        pltpu.make_async_copy(
            outb[0], out_hbm.at[pl.ds(lastb, CHUNK)], semO[0]).wait()
        pltpu.make_async_copy(
            outb[1], out_hbm.at[pl.ds(lastb, CHUNK)], semO[1]).wait()

    return enc


@jax.jit
def kernel(x, plane_params, line_params):
    npts = x.shape[0]
    planes_t = plane_params.transpose(0, 2, 3, 1).reshape(3, RES * RES, NCH)
    lines_t = line_params[..., 0].transpose(0, 2, 1).reshape(3 * RES * NCH)
    return _build(npts)(x, planes_t, lines_t)


# speedup vs baseline: 1.3436x; 1.3436x over previous
import functools

import jax
import jax.numpy as jnp
from jax import lax
from jax.experimental import pallas as pl
from jax.experimental.pallas import tpu as pltpu
from jax.experimental.pallas import tpu_sc as plsc

RES = 512
NCH = 32
LANES = 16
CHUNK = 128
GROUPS = CHUNK // LANES
NCORES = 2
NSUB = 16
NW = NCORES * NSUB

PLANE_COLS = ((0, 1), (0, 2), (1, 2))


@functools.lru_cache(maxsize=None)
def _build(npts):
    per_w = npts // NW
    nchunks = per_w // CHUNK
    assert nchunks % 2 == 0
    mesh = plsc.VectorSubcoreMesh(core_axis_name="c", subcore_axis_name="s")

    @functools.partial(
        pl.kernel,
        mesh=mesh,
        out_type=jax.ShapeDtypeStruct((npts, 3 * NCH), jnp.float32),
        compiler_params=pltpu.CompilerParams(
            needs_layout_passes=False, use_tc_tiling_on_sc=False),
        scratch_types=(
            [pltpu.VMEM((3 * RES * NCH,), jnp.float32)]
            + [pltpu.VMEM((CHUNK, 3), jnp.float32)
               for _ in range(2)]
            + [pltpu.VMEM((CHUNK,), jnp.int32)
               for _ in range(24)]
            + [pltpu.VMEM((CHUNK,), jnp.float32)
               for _ in range(36)]
            + [pltpu.VMEM((CHUNK,), jnp.int32)
               for _ in range(6)]
            + [pltpu.VMEM((CHUNK, NCH), jnp.float32)
               for _ in range(8)]
            + [pltpu.VMEM((CHUNK, 3 * NCH), jnp.float32)
               for _ in range(2)]
            + [pltpu.SemaphoreType.DMA] * 6
        ),
    )
    def enc(x_hbm, planes_hbm, lines_hbm, out_hbm, *scratch):
        line_v = scratch[0]
        xbuf = [scratch[1 + j] for j in range(2)]
        idx = [scratch[3 + j] for j in range(24)]
        wgt = [scratch[27 + j] for j in range(36)]
        lbv = [scratch[63 + j] for j in range(6)]
        rbuf = [scratch[69 + j] for j in range(8)]
        outb = [scratch[77 + j] for j in range(2)]
        semA, semB, semX0, semX1, semO0, semO1 = scratch[79:85]
        semX = [semX0, semX1]
        semO = [semO0, semO1]

        wid = lax.axis_index("s") * NCORES + lax.axis_index("c")
        wbase = wid * per_w
        pltpu.sync_copy(lines_hbm, line_v)
        iota = lax.iota(jnp.int32, LANES)

        def fire_x(ci, par):
            return pltpu.async_copy(
                x_hbm.at[pl.ds(wbase + ci * CHUNK, CHUNK)], xbuf[par],
                semX[par])

        def fire_planes(k, s, ab, sem):
            bufs = rbuf[ab * 4:ab * 4 + 4]
            return [pltpu.async_copy(
                planes_hbm.at[k].at[idx[s * 12 + k * 4 + j]],
                bufs[j], sem) for j in range(4)]

        def wait_planes(k, s, ab, sem):
            bufs = rbuf[ab * 4:ab * 4 + 4]
            for j in range(4):
                pltpu.make_async_copy(
                    planes_hbm.at[k].at[idx[s * 12 + k * 4 + j]], bufs[j],
                    sem).wait()

        def grp_idx(s, xv):
            def body(g, _):
                rows = g * LANES + iota
                sl = pl.ds(g * LANES, LANES)
                i0 = []
                fr = []
                for col in range(3):
                    xc = plsc.load_gather(
                        xv, [rows, jnp.full((LANES,), col, jnp.int32)])
                    ic = xc * float(RES - 1)
                    ic0 = jnp.minimum(ic.astype(jnp.int32), RES - 2)
                    i0.append(ic0)
                    fr.append(ic - ic0.astype(jnp.float32))
                for k in range(3):
                    axc, ayc = PLANE_COLS[k]
                    fx, fy, fl = fr[axc], fr[ayc], fr[2 - k]
                    gx = 1.0 - fx
                    gy = 1.0 - fy
                    wgt[s * 18 + k * 6 + 0][sl] = gx * gy
                    wgt[s * 18 + k * 6 + 1][sl] = fx * gy
                    wgt[s * 18 + k * 6 + 2][sl] = gx * fy
                    wgt[s * 18 + k * 6 + 3][sl] = fx * fy
                    wgt[s * 18 + k * 6 + 4][sl] = 1.0 - fl
                    wgt[s * 18 + k * 6 + 5][sl] = fl
                    row00 = i0[ayc] * RES + i0[axc]
                    idx[s * 12 + k * 4 + 0][sl] = row00
                    idx[s * 12 + k * 4 + 1][sl] = row00 + 1
                    idx[s * 12 + k * 4 + 2][sl] = row00 + RES
                    idx[s * 12 + k * 4 + 3][sl] = row00 + RES + 1
                    lbv[s * 3 + k][sl] = (k * RES + i0[2 - k]) * NCH
                return 0

            lax.fori_loop(0, GROUPS, body, 0)

        def compute(k, s, ab, out_v):
            r00, r01, r10, r11 = rbuf[ab * 4:ab * 4 + 4]
            w = wgt[s * 18 + k * 6:s * 18 + k * 6 + 6]
            lb_v = lbv[s * 3 + k]

            @plsc.parallel_loop(0, GROUPS, unroll=2)
            def grp(g):
                gbase = g * LANES
                gsl = pl.ds(gbase, LANES)
                wg = [w[i][gsl] for i in range(6)]
                lbg = lb_v[gsl]
                for j in range(LANES):
                    p = gbase + j
                    w00 = jnp.full((LANES,), wg[0][j], jnp.float32)
                    w01 = jnp.full((LANES,), wg[1][j], jnp.float32)
                    w10 = jnp.full((LANES,), wg[2][j], jnp.float32)
                    w11 = jnp.full((LANES,), wg[3][j], jnp.float32)
                    wl0 = jnp.full((LANES,), wg[4][j], jnp.float32)
                    wl1 = jnp.full((LANES,), wg[5][j], jnp.float32)
                    lb = lbg[j]
                    for h in range(2):
                        off = h * LANES
                        sl = pl.ds(off, LANES)
                        v00 = r00[p, sl]
                        v01 = r01[p, sl]
                        v10 = r10[p, sl]
                        v11 = r11[p, sl]
                        l0 = line_v[pl.ds(lb + off, LANES)]
                        l1 = line_v[pl.ds(lb + NCH + off, LANES)]
                        pf = w00 * v00 + w01 * v01 + w10 * v10 + w11 * v11
                        lf = wl0 * l0 + wl1 * l1
                        out_v[p, pl.ds(k * NCH + off, LANES)] = pf * lf

        fire_x(0, 0).wait()
        grp_idx(0, xbuf[0])
        fire_planes(0, 0, 0, semA)
        fire_planes(1, 0, 1, semB)
        fire_x(1, 1)

        def half(ci, par):
            s = par
            nxt = jnp.minimum(ci + 1, nchunks - 1)
            base = wbase + ci * CHUNK
            out_v = outb[par]

            @pl.when(ci >= 2)
            def _():
                pltpu.make_async_copy(
                    out_v, out_hbm.at[pl.ds(base, CHUNK)], semO[par]).wait()

            wait_planes(0, s, 0, semA)
            compute(0, s, 0, out_v)
            fire_planes(2, s, 0, semA)
            wait_planes(1, s, 1, semB)
            compute(1, s, 1, out_v)
            wait_planes(2, s, 0, semA)
            compute(2, s, 0, out_v)

            pltpu.make_async_copy(
                x_hbm.at[pl.ds(wbase + nxt * CHUNK, CHUNK)], xbuf[1 - par],
                semX[1 - par]).wait()
            grp_idx(1 - s, xbuf[1 - par])
            fire_planes(0, 1 - s, 0, semA)
            fire_planes(1, 1 - s, 1, semB)
            nxt2 = jnp.minimum(ci + 2, nchunks - 1)
            fire_x(nxt2, par)
            pltpu.async_copy(out_v, out_hbm.at[pl.ds(base, CHUNK)], semO[par])

        def pair(pi, _):
            half(2 * pi, 0)
            half(2 * pi + 1, 1)
            return 0

        lax.fori_loop(0, nchunks // 2, pair, 0)

        lastb = wbase + (nchunks - 1) * CHUNK
        wait_planes(0, 0, 0, semA)
        wait_planes(1, 0, 1, semB)
        pltpu.make_async_copy(
            x_hbm.at[pl.ds(lastb, CHUNK)], xbuf[1], semX[1]).wait()
        pltpu.make_async_copy(
            outb[0], out_hbm.at[pl.ds(lastb, CHUNK)], semO[0]).wait()
        pltpu.make_async_copy(
            outb[1], out_hbm.at[pl.ds(lastb, CHUNK)], semO[1]).wait()

    return enc


@jax.jit
def kernel(x, plane_params, line_params):
    npts = x.shape[0]
    planes_t = plane_params.transpose(0, 2, 3, 1).reshape(3, RES * RES, NCH)
    lines_t = line_params[..., 0].transpose(0, 2, 1).reshape(3 * RES * NCH)
    return _build(npts)(x, planes_t, lines_t)
